# Initial kernel scaffold; baseline (speedup 1.0000x reference)
import jax
import jax.numpy as jnp
from jax.experimental import pallas as pl


def kernel(x, edge_index, node_to_graph_map, W_in, b_in, W_msg, b_msg, W_ex, b_ex):
    raise NotImplementedError("write your pallas kernel here")


# speedup vs baseline: 1.9508x; 1.9508x over previous
import functools

import jax
import jax.numpy as jnp
from jax import lax
from jax.experimental import pallas as pl
from jax.experimental.pallas import tpu as pltpu
from jax.experimental.pallas import tpu_sc as plsc

N = 10000
D = 256
E = 160000
H = 256
G = 32

RB = 1000
NB = N // RB
HH = H // 2

NTILES = 16
CHUNK = 128
CHUNKS_PER_TILE = 80
EPT = CHUNK * CHUNKS_PER_TILE
EPAD = EPT * NTILES
PH = 5120
ACC_ROWS = 5248
ZR = ACC_ROWS // NTILES
OUTR = PH // NTILES


def _node_proj_body(x_ref, win_ref, bin_ref, wm_ref, bm_ref, a_ref, b_ref, h0_s):
    ch = pl.program_id(1)

    @pl.when(ch == 0)
    def _():
        h0_s[...] = (
            jnp.dot(x_ref[...], win_ref[...], preferred_element_type=jnp.float32)
            + bin_ref[...]
        )

    h0 = h0_s[...]
    wc = wm_ref[...]
    a_ref[...] = jnp.dot(h0, wc[0:H, :], preferred_element_type=jnp.float32)
    b_ref[...] = (
        jnp.dot(h0, wc[H : 2 * H, :], preferred_element_type=jnp.float32)
        + bm_ref[0]
    )


def _node_proj(x, W_in, b_in2, wmh, bmh):
    return pl.pallas_call(
        _node_proj_body,
        grid=(NB, 2),
        in_specs=[
            pl.BlockSpec((RB, D), lambda i, ch: (i, 0)),
            pl.BlockSpec((D, H), lambda i, ch: (0, 0)),
            pl.BlockSpec((1, H), lambda i, ch: (0, 0)),
            pl.BlockSpec((2 * H, HH), lambda i, ch: (ch, 0)),
            pl.BlockSpec((1, 1, HH), lambda i, ch: (ch, 0, 0)),
        ],
        out_specs=[
            pl.BlockSpec((RB, HH), lambda i, ch: (ch * NB + i, 0)),
            pl.BlockSpec((RB, HH), lambda i, ch: (ch * NB + i, 0)),
        ],
        out_shape=[
            jax.ShapeDtypeStruct((2 * N, HH), jnp.float32),
            jax.ShapeDtypeStruct((2 * N, HH), jnp.float32),
        ],
        scratch_shapes=[pltpu.VMEM((RB, H), jnp.float32)],
    )(x, W_in, b_in2, wmh, bmh)


@functools.cache
def _make_edge_agg():
    mesh = plsc.VectorSubcoreMesh(core_axis_name="c", subcore_axis_name="s")
    return pl.kernel(
        _edge_agg_body,
        out_type=jax.ShapeDtypeStruct((2, 2 * PH, HH), jnp.float32),
        mesh=mesh,
        compiler_params=pltpu.CompilerParams(needs_layout_passes=False),
        scratch_types=[
            pltpu.VMEM((CHUNKS_PER_TILE, CHUNK), jnp.int32),
            pltpu.VMEM((CHUNKS_PER_TILE, CHUNK), jnp.int32),
            pltpu.VMEM((EPT,), jnp.int32),
            pltpu.VMEM((EPT,), jnp.int32),
            pltpu.VMEM((1, CHUNK), jnp.int32),
            pltpu.VMEM((1, CHUNK), jnp.int32),
            pltpu.VMEM((1, CHUNK), jnp.int32),
            pltpu.VMEM((CHUNK, HH), jnp.float32),
            pltpu.VMEM((CHUNK, HH), jnp.float32),
            pltpu.VMEM_SHARED((ACC_ROWS, HH), jnp.float32),
            pltpu.SemaphoreType.DMA,
        ],
    )


def _edge_agg_body(a2_hbm, b2_hbm, src_hbm, dst_hbm, z_hbm, out_hbm,
                   src_o, dst_o, csrc, cdst, tsrc, tdstg, tdsts,
                   buf_a, buf_b, acc_sh, sem):
    c = lax.axis_index("c")
    s = lax.axis_index("s")
    row0 = s * CHUNKS_PER_TILE
    coff = c * N

    pltpu.sync_copy(src_hbm.at[pl.ds(row0, CHUNKS_PER_TILE)], src_o)
    pltpu.sync_copy(dst_hbm.at[pl.ds(row0, CHUNKS_PER_TILE)], dst_o)

    for p in (0, 1):
        lo = p * PH
        fill = (p + 1) * PH

        def _pre(r, carry):
            base = r * CHUNK
            for v in range(CHUNK // 16):
                sl = pl.ds(base + v * 16, 16)
                csrc[sl] = jnp.zeros((16,), jnp.int32)
                cdst[sl] = jnp.full((16,), fill, jnp.int32)
            return carry

        lax.fori_loop(0, CHUNKS_PER_TILE, _pre, 0)

        def _compact(r, o):
            for v in range(CHUNK // 16):
                sl = pl.ds(v * 16, 16)
                d = dst_o[r, sl]
                sv = src_o[r, sl]
                m = (d >= lo) & (d < lo + PH)
                mc = m.astype(jnp.int32)
                pos = plsc.cumsum(mc) + (o - 1)
                plsc.store_scatter(cdst, [pos], d, mask=m)
                plsc.store_scatter(csrc, [pos], sv, mask=m)
                o = o + jnp.sum(mc)
            return o

        cnt = lax.fori_loop(0, CHUNKS_PER_TILE, _compact, 0)
        nchunks = (cnt + CHUNK - 1) // CHUNK

        pltpu.sync_copy(z_hbm, acc_sh.at[pl.ds(s * ZR, ZR)])
        plsc.subcore_barrier()

        def _chunk(j, carry):
            base = j * CHUNK
            for v in range(CHUNK // 16):
                slc = pl.ds(base + v * 16, 16)
                slt = pl.ds(v * 16, 16)
                sv = csrc[slc]
                dv = cdst[slc]
                tsrc[0, slt] = sv + coff
                tdstg[0, slt] = jnp.minimum(dv, N - 1) + coff
                tdsts[0, slt] = dv - lo
            pltpu.async_copy(a2_hbm.at[tsrc.at[0]], buf_a, sem).wait()
            pltpu.async_copy(b2_hbm.at[tdstg.at[0]], buf_b, sem).wait()

            def _row(r, rc):
                for v in range(HH // 16):
                    sl = pl.ds(v * 16, 16)
                    buf_a[r, sl] = jnp.maximum(buf_a[r, sl] + buf_b[r, sl], 0.0)
                return rc

            lax.fori_loop(0, CHUNK, _row, 0)
            pltpu.sync_copy(buf_a, acc_sh.at[tdsts.at[0]], add=True)
            return carry

        lax.fori_loop(0, nchunks, _chunk, 0)
        plsc.subcore_barrier()

        pltpu.sync_copy(
            acc_sh.at[pl.ds(s * OUTR, OUTR)],
            out_hbm.at[c, pl.ds(p * PH + s * OUTR, OUTR)],
        )
        if p == 0:
            plsc.subcore_barrier()


def _graph_stats_body(a0_ref, a1_ref, m_ref, gs_ref, cnt_ref):
    i = pl.program_id(0)
    m = m_ref[0, 0, :]
    oh = (lax.broadcasted_iota(jnp.int32, (G, RB), 0) == m[None, :]).astype(
        jnp.float32
    )
    h = jnp.maximum(jnp.concatenate([a0_ref[...], a1_ref[...]], axis=1), 0.0)
    gs = jnp.dot(oh, h, preferred_element_type=jnp.float32)
    cnt = jnp.broadcast_to(jnp.sum(oh, axis=1, keepdims=True), (G, HH))

    @pl.when(i == 0)
    def _():
        gs_ref[...] = gs
        cnt_ref[...] = cnt

    @pl.when(i > 0)
    def _():
        gs_ref[...] += gs
        cnt_ref[...] += cnt


def _graph_stats(agg0, agg1, map3):
    hspec = pl.BlockSpec((RB, HH), lambda i: (i, 0))
    return pl.pallas_call(
        _graph_stats_body,
        grid=(NB,),
        in_specs=[
            hspec,
            hspec,
            pl.BlockSpec((1, 1, RB), lambda i: (i, 0, 0)),
        ],
        out_specs=[
            pl.BlockSpec((G, H), lambda i: (0, 0)),
            pl.BlockSpec((G, HH), lambda i: (0, 0)),
        ],
        out_shape=[
            jax.ShapeDtypeStruct((G, H), jnp.float32),
            jax.ShapeDtypeStruct((G, HH), jnp.float32),
        ],
    )(agg0, agg1, map3)


def _final_mix_body(a0_ref, a1_ref, m_ref, gs_ref, cnt_ref, wex_ref, bex_ref,
                    out_ref):
    cnt = jnp.maximum(cnt_ref[...], 1.0)
    gsum = gs_ref[...]
    mean = jnp.concatenate([gsum[:, :HH] / cnt, gsum[:, HH:] / cnt], axis=1)
    wex = wex_ref[...]
    mixg = (
        jnp.dot(mean, wex[H : 2 * H, :], preferred_element_type=jnp.float32)
        + bex_ref[...]
    )
    h = jnp.maximum(jnp.concatenate([a0_ref[...], a1_ref[...]], axis=1), 0.0)
    t1 = jnp.dot(h, wex[0:H, :], preferred_element_type=jnp.float32)
    m = m_ref[0, 0, :]
    ohn = (m[:, None] == lax.broadcasted_iota(jnp.int32, (RB, G), 1)).astype(
        jnp.float32
    )
    g_rows = jnp.dot(ohn, mixg, preferred_element_type=jnp.float32)
    exch = jnp.tanh(t1 + g_rows)
    out_ref[...] = h + exch


def _final_mix(agg0, agg1, map3, gsum, cnt, W_ex, b_ex2):
    hspec = pl.BlockSpec((RB, HH), lambda i: (i, 0))
    return pl.pallas_call(
        _final_mix_body,
        grid=(NB,),
        in_specs=[
            hspec,
            hspec,
            pl.BlockSpec((1, 1, RB), lambda i: (i, 0, 0)),
            pl.BlockSpec((G, H), lambda i: (0, 0)),
            pl.BlockSpec((G, HH), lambda i: (0, 0)),
            pl.BlockSpec((2 * H, H), lambda i: (0, 0)),
            pl.BlockSpec((1, H), lambda i: (0, 0)),
        ],
        out_specs=pl.BlockSpec((RB, H), lambda i: (i, 0)),
        out_shape=jax.ShapeDtypeStruct((N, H), jnp.float32),
    )(agg0, agg1, map3, gsum, cnt, W_ex, b_ex2)


def kernel(x, edge_index, node_to_graph_map, W_in, b_in, W_msg, b_msg, W_ex, b_ex):
    wmh = jnp.concatenate([W_msg[:, :HH], W_msg[:, HH:]], axis=0)
    bmh = b_msg.reshape(2, 1, HH)
    a2, b2 = _node_proj(x, W_in, b_in.reshape(1, H), wmh, bmh)

    src = edge_index[0]
    dst = edge_index[1]
    pad = EPAD - E
    src_p = jnp.concatenate([src, jnp.zeros((pad,), jnp.int32)]).reshape(
        EPAD // CHUNK, CHUNK
    )
    dst_p = jnp.concatenate([dst, jnp.full((pad,), N, jnp.int32)]).reshape(
        EPAD // CHUNK, CHUNK
    )
    zrows = jnp.zeros((ZR, HH), jnp.float32)

    agg = _make_edge_agg()(a2, b2, src_p, dst_p, zrows)
    agg0 = agg[0, :N]
    agg1 = agg[1, :N]

    map3 = node_to_graph_map.reshape(NB, 1, RB)
    gsum, cnt = _graph_stats(agg0, agg1, map3)
    return _final_mix(agg0, agg1, map3, gsum, cnt, W_ex, b_ex.reshape(1, H))


# speedup vs baseline: 2.5168x; 1.2902x over previous
import functools

import jax
import jax.numpy as jnp
from jax import lax
from jax.experimental import pallas as pl
from jax.experimental.pallas import tpu as pltpu
from jax.experimental.pallas import tpu_sc as plsc

N = 10000
D = 256
E = 160000
H = 256
G = 32

RB = 1000
NB = N // RB
HH = H // 2

NTILES = 16
CHUNK = 128
CHUNKS_PER_TILE = 80
EPT = CHUNK * CHUNKS_PER_TILE
EPAD = EPT * NTILES
PH = 5120
ACC_ROWS = 5248
ZR = ACC_ROWS // NTILES
OUTR = PH // NTILES


def _node_proj_body(x_ref, win_ref, bin_ref, wm_ref, bm_ref, a_ref, b_ref, h0_s):
    ch = pl.program_id(1)

    @pl.when(ch == 0)
    def _():
        h0_s[...] = (
            jnp.dot(x_ref[...], win_ref[...], preferred_element_type=jnp.float32)
            + bin_ref[...]
        )

    h0 = h0_s[...]
    wc = wm_ref[...]
    a_ref[...] = jnp.dot(h0, wc[0:H, :], preferred_element_type=jnp.float32)
    b_ref[...] = (
        jnp.dot(h0, wc[H : 2 * H, :], preferred_element_type=jnp.float32)
        + bm_ref[0]
    )


def _node_proj(x, W_in, b_in2, wmh, bmh):
    return pl.pallas_call(
        _node_proj_body,
        grid=(NB, 2),
        in_specs=[
            pl.BlockSpec((RB, D), lambda i, ch: (i, 0)),
            pl.BlockSpec((D, H), lambda i, ch: (0, 0)),
            pl.BlockSpec((1, H), lambda i, ch: (0, 0)),
            pl.BlockSpec((2 * H, HH), lambda i, ch: (ch, 0)),
            pl.BlockSpec((1, 1, HH), lambda i, ch: (ch, 0, 0)),
        ],
        out_specs=[
            pl.BlockSpec((RB, HH), lambda i, ch: (ch * NB + i, 0)),
            pl.BlockSpec((RB, HH), lambda i, ch: (ch * NB + i, 0)),
        ],
        out_shape=[
            jax.ShapeDtypeStruct((2 * N, HH), jnp.float32),
            jax.ShapeDtypeStruct((2 * N, HH), jnp.float32),
        ],
        scratch_shapes=[pltpu.VMEM((RB, H), jnp.float32)],
    )(x, W_in, b_in2, wmh, bmh)


@functools.cache
def _make_edge_agg():
    mesh = plsc.VectorSubcoreMesh(core_axis_name="c", subcore_axis_name="s")
    return pl.kernel(
        _edge_agg_body,
        out_type=jax.ShapeDtypeStruct((2, 2 * PH, HH), jnp.float32),
        mesh=mesh,
        compiler_params=pltpu.CompilerParams(needs_layout_passes=False),
        scratch_types=[
            pltpu.VMEM((CHUNKS_PER_TILE, CHUNK), jnp.int32),
            pltpu.VMEM((CHUNKS_PER_TILE, CHUNK), jnp.int32),
            pltpu.VMEM((EPT,), jnp.int32),
            pltpu.VMEM((EPT,), jnp.int32),
            pltpu.VMEM((1, CHUNK), jnp.int32),
            pltpu.VMEM((1, CHUNK), jnp.int32),
            pltpu.VMEM((1, CHUNK), jnp.int32),
            pltpu.VMEM((1, CHUNK), jnp.int32),
            pltpu.VMEM((1, CHUNK), jnp.int32),
            pltpu.VMEM((1, CHUNK), jnp.int32),
            pltpu.VMEM((CHUNK, HH), jnp.float32),
            pltpu.VMEM((CHUNK, HH), jnp.float32),
            pltpu.VMEM_SHARED((ACC_ROWS, HH), jnp.float32),
            pltpu.SemaphoreType.DMA,
            pltpu.SemaphoreType.DMA,
            pltpu.SemaphoreType.DMA,
            pltpu.SemaphoreType.DMA,
        ],
    )


def _edge_agg_body(a2_hbm, b2_hbm, src_hbm, dst_hbm, z_hbm, out_hbm,
                   src_o, dst_o, csrc, cdst,
                   tsrc0, tdstg0, tdsts0, tsrc1, tdstg1, tdsts1,
                   buf0, buf1, acc_sh, sema0, sema1, semb0, semb1):
    c = lax.axis_index("c")
    s = lax.axis_index("s")
    row0 = s * CHUNKS_PER_TILE
    coff = c * N

    pltpu.sync_copy(src_hbm.at[pl.ds(row0, CHUNKS_PER_TILE)], src_o)
    pltpu.sync_copy(dst_hbm.at[pl.ds(row0, CHUNKS_PER_TILE)], dst_o)

    for p in (0, 1):
        lo = p * PH
        fill = (p + 1) * PH

        def _pre(r, carry):
            base = r * CHUNK
            for v in range(CHUNK // 16):
                sl = pl.ds(base + v * 16, 16)
                csrc[sl] = jnp.zeros((16,), jnp.int32)
                cdst[sl] = jnp.full((16,), fill, jnp.int32)
            return carry

        lax.fori_loop(0, CHUNKS_PER_TILE, _pre, 0)

        def _compact(r, o):
            for v in range(CHUNK // 16):
                sl = pl.ds(v * 16, 16)
                d = dst_o[r, sl]
                sv = src_o[r, sl]
                m = (d >= lo) & (d < lo + PH)
                mc = m.astype(jnp.int32)
                pos = plsc.cumsum(mc) + (o - 1)
                plsc.store_scatter(cdst, [pos], d, mask=m)
                plsc.store_scatter(csrc, [pos], sv, mask=m)
                o = o + jnp.sum(mc)
            return o

        cnt = lax.fori_loop(0, CHUNKS_PER_TILE, _compact, 0)
        nchunks = (cnt + CHUNK - 1) // CHUNK
        nch2 = ((nchunks + 1) // 2) * 2

        pltpu.sync_copy(z_hbm, acc_sh.at[pl.ds(s * ZR, ZR)])
        plsc.subcore_barrier()

        def _mk_idx(j, tsrc, tdstg, tdsts):
            base = j * CHUNK
            for v in range(CHUNK // 16):
                slc = pl.ds(base + v * 16, 16)
                slt = pl.ds(v * 16, 16)
                sv = csrc[slc]
                dv = cdst[slc]
                tsrc[0, slt] = sv + coff
                tdstg[0, slt] = jnp.minimum(dv, N - 1) + coff
                tdsts[0, slt] = dv - lo

        def _issue_a(tsrc, buf, sema):
            pltpu.async_copy(a2_hbm.at[tsrc.at[0]], buf, sema)

        def _drain_a(buf, sema):
            pltpu.make_async_copy(a2_hbm.at[pl.ds(0, CHUNK)], buf, sema).wait()

        def _relu_scatter(buf, tdsts):
            def _row(r, rc):
                for v in range(HH // 16):
                    sl = pl.ds(v * 16, 16)
                    buf[r, sl] = jnp.maximum(buf[r, sl], 0.0)
                return rc

            lax.fori_loop(0, CHUNK, _row, 0)
            pltpu.sync_copy(buf, acc_sh.at[tdsts.at[0]], add=True)

        @pl.when(nch2 > 0)
        def _():
            _mk_idx(0, tsrc0, tdstg0, tdsts0)
            _issue_a(tsrc0, buf0, sema0)
            _mk_idx(1, tsrc1, tdstg1, tdsts1)
            _issue_a(tsrc1, buf1, sema1)

        def _pipe(i2, carry):
            j0 = 2 * i2
            _drain_a(buf0, sema0)
            db0 = pltpu.async_copy(b2_hbm.at[tdstg0.at[0]], buf0, semb0, add=True)
            _drain_a(buf1, sema1)
            db1 = pltpu.async_copy(b2_hbm.at[tdstg1.at[0]], buf1, semb1, add=True)
            db0.wait()
            _relu_scatter(buf0, tdsts0)

            @pl.when(j0 + 2 < nch2)
            def _():
                _mk_idx(j0 + 2, tsrc0, tdstg0, tdsts0)
                _issue_a(tsrc0, buf0, sema0)

            db1.wait()
            _relu_scatter(buf1, tdsts1)

            @pl.when(j0 + 3 < nch2)
            def _():
                _mk_idx(j0 + 3, tsrc1, tdstg1, tdsts1)
                _issue_a(tsrc1, buf1, sema1)

            return carry

        lax.fori_loop(0, nch2 // 2, _pipe, 0)
        plsc.subcore_barrier()

        pltpu.sync_copy(
            acc_sh.at[pl.ds(s * OUTR, OUTR)],
            out_hbm.at[c, pl.ds(p * PH + s * OUTR, OUTR)],
        )
        if p == 0:
            plsc.subcore_barrier()


def _graph_stats_body(a0_ref, a1_ref, m_ref, gs_ref, cnt_ref):
    i = pl.program_id(0)
    m = m_ref[0, 0, :]
    oh = (lax.broadcasted_iota(jnp.int32, (G, RB), 0) == m[None, :]).astype(
        jnp.float32
    )
    h = jnp.maximum(jnp.concatenate([a0_ref[...], a1_ref[...]], axis=1), 0.0)
    gs = jnp.dot(oh, h, preferred_element_type=jnp.float32)
    cnt = jnp.broadcast_to(jnp.sum(oh, axis=1, keepdims=True), (G, HH))

    @pl.when(i == 0)
    def _():
        gs_ref[...] = gs
        cnt_ref[...] = cnt

    @pl.when(i > 0)
    def _():
        gs_ref[...] += gs
        cnt_ref[...] += cnt


def _graph_stats(agg0, agg1, map3):
    hspec = pl.BlockSpec((RB, HH), lambda i: (i, 0))
    return pl.pallas_call(
        _graph_stats_body,
        grid=(NB,),
        in_specs=[
            hspec,
            hspec,
            pl.BlockSpec((1, 1, RB), lambda i: (i, 0, 0)),
        ],
        out_specs=[
            pl.BlockSpec((G, H), lambda i: (0, 0)),
            pl.BlockSpec((G, HH), lambda i: (0, 0)),
        ],
        out_shape=[
            jax.ShapeDtypeStruct((G, H), jnp.float32),
            jax.ShapeDtypeStruct((G, HH), jnp.float32),
        ],
    )(agg0, agg1, map3)


def _final_mix_body(a0_ref, a1_ref, m_ref, gs_ref, cnt_ref, wex_ref, bex_ref,
                    out_ref):
    cnt = jnp.maximum(cnt_ref[...], 1.0)
    gsum = gs_ref[...]
    mean = jnp.concatenate([gsum[:, :HH] / cnt, gsum[:, HH:] / cnt], axis=1)
    wex = wex_ref[...]
    mixg = (
        jnp.dot(mean, wex[H : 2 * H, :], preferred_element_type=jnp.float32)
        + bex_ref[...]
    )
    h = jnp.maximum(jnp.concatenate([a0_ref[...], a1_ref[...]], axis=1), 0.0)
    t1 = jnp.dot(h, wex[0:H, :], preferred_element_type=jnp.float32)
    m = m_ref[0, 0, :]
    ohn = (m[:, None] == lax.broadcasted_iota(jnp.int32, (RB, G), 1)).astype(
        jnp.float32
    )
    g_rows = jnp.dot(ohn, mixg, preferred_element_type=jnp.float32)
    exch = jnp.tanh(t1 + g_rows)
    out_ref[...] = h + exch


def _final_mix(agg0, agg1, map3, gsum, cnt, W_ex, b_ex2):
    hspec = pl.BlockSpec((RB, HH), lambda i: (i, 0))
    return pl.pallas_call(
        _final_mix_body,
        grid=(NB,),
        in_specs=[
            hspec,
            hspec,
            pl.BlockSpec((1, 1, RB), lambda i: (i, 0, 0)),
            pl.BlockSpec((G, H), lambda i: (0, 0)),
            pl.BlockSpec((G, HH), lambda i: (0, 0)),
            pl.BlockSpec((2 * H, H), lambda i: (0, 0)),
            pl.BlockSpec((1, H), lambda i: (0, 0)),
        ],
        out_specs=pl.BlockSpec((RB, H), lambda i: (i, 0)),
        out_shape=jax.ShapeDtypeStruct((N, H), jnp.float32),
    )(agg0, agg1, map3, gsum, cnt, W_ex, b_ex2)


def kernel(x, edge_index, node_to_graph_map, W_in, b_in, W_msg, b_msg, W_ex, b_ex):
    wmh = jnp.concatenate([W_msg[:, :HH], W_msg[:, HH:]], axis=0)
    bmh = b_msg.reshape(2, 1, HH)
    a2, b2 = _node_proj(x, W_in, b_in.reshape(1, H), wmh, bmh)

    src = edge_index[0]
    dst = edge_index[1]
    pad = EPAD - E
    src_p = jnp.concatenate([src, jnp.zeros((pad,), jnp.int32)]).reshape(
        EPAD // CHUNK, CHUNK
    )
    dst_p = jnp.concatenate([dst, jnp.full((pad,), N, jnp.int32)]).reshape(
        EPAD // CHUNK, CHUNK
    )
    zrows = jnp.zeros((ZR, HH), jnp.float32)

    agg = _make_edge_agg()(a2, b2, src_p, dst_p, zrows)
    agg0 = agg[0, :N]
    agg1 = agg[1, :N]

    map3 = node_to_graph_map.reshape(NB, 1, RB)
    gsum, cnt = _graph_stats(agg0, agg1, map3)
    return _final_mix(agg0, agg1, map3, gsum, cnt, W_ex, b_ex.reshape(1, H))
